# baseline (speedup 1.0000x reference)
import functools

import jax
import jax.numpy as jnp
from jax import lax
from jax.experimental import pallas as pl
from jax.experimental.pallas import tpu as pltpu
from jax.experimental.pallas import tpu_sc as plsc

N = 10000
E = 320000
D = 128
H = 128
L = 3
G = 64
OUT = 10

NC = 2
NS = 16
NW = NC * NS
EPW = E // NW
CHUNK = 80
NCHUNK = EPW // CHUNK
NBUF = 4
ZROWS_BIG = 632
ZROWS_SMALL = 624

@functools.cache
def _make_sc_aggregate():
    mesh = plsc.VectorSubcoreMesh(core_axis_name="c", subcore_axis_name="s")
    return functools.partial(
        pl.kernel,
        mesh=mesh,
        out_type=jax.ShapeDtypeStruct((NC, N, H), jnp.float32),
        scratch_types=(
            [pltpu.VMEM_SHARED((N, H), jnp.float32)]
            + [pltpu.VMEM((CHUNK, H), jnp.float32)] * NBUF
            + [pltpu.VMEM((CHUNK,), jnp.int32)] * NBUF
            + [pltpu.VMEM((CHUNK,), jnp.int32)] * NBUF
            + [pltpu.SemaphoreType.DMA] * (3 * NBUF)
        ),
    )(_sc_aggregate_body)


def _sc_aggregate_body(x_hbm, src_hbm, dst_hbm, out_hbm,
                       acc_sh, *bufs_and_sems):
    rows = bufs_and_sems[:NBUF]
    srcb = bufs_and_sems[NBUF:2 * NBUF]
    dstb = bufs_and_sems[2 * NBUF:3 * NBUF]
    gsem = bufs_and_sems[3 * NBUF:4 * NBUF]
    ssem = bufs_and_sems[4 * NBUF:5 * NBUF]
    isem = bufs_and_sems[5 * NBUF:]
    c = lax.axis_index("c")
    s = lax.axis_index("s")
    wid = c * NS + s

    z16 = jnp.zeros((16,), jnp.float32)

    def _zero_body(i, carry):
        r = i // (H // 16)
        col = (i % (H // 16)) * 16
        rows[0][r, pl.ds(col, 16)] = z16
        return carry

    lax.fori_loop(0, CHUNK * (H // 16), _zero_body, 0)

    start_big = s * ZROWS_BIG
    start_small = 2 * ZROWS_BIG + (s - 2) * ZROWS_SMALL
    start = jnp.where(s < 2, start_big, start_small)
    nzfull = ZROWS_SMALL // CHUNK

    @pl.when(c == 0)
    def _():
        @pl.when(s < 2)
        def _():
            pltpu.sync_copy(x_hbm.at[pl.ds(s * ZROWS_BIG, ZROWS_BIG)],
                            acc_sh.at[pl.ds(s * ZROWS_BIG, ZROWS_BIG)])

        @pl.when(s >= 2)
        def _():
            d0 = 2 * ZROWS_BIG + (s - 2) * ZROWS_SMALL
            pltpu.sync_copy(x_hbm.at[pl.ds(d0, ZROWS_SMALL)],
                            acc_sh.at[pl.ds(d0, ZROWS_SMALL)])

    @pl.when(c == 1)
    def _():
        for k in range(nzfull):
            pltpu.async_copy(rows[0],
                             acc_sh.at[pl.ds(start + k * CHUNK, CHUNK)],
                             gsem[0])
        nfull = nzfull * CHUNK

        @pl.when(s < 2)
        def _():
            pltpu.async_copy(rows[0].at[pl.ds(0, ZROWS_BIG - nfull)],
                             acc_sh.at[pl.ds(start + nfull, ZROWS_BIG - nfull)],
                             gsem[0])

        @pl.when(s >= 2)
        def _():
            pltpu.async_copy(rows[0].at[pl.ds(0, ZROWS_SMALL - nfull)],
                             acc_sh.at[pl.ds(start + nfull,
                                             ZROWS_SMALL - nfull)],
                             gsem[0])

        for k in range(nzfull):
            pltpu.make_async_copy(rows[0], acc_sh.at[pl.ds(0, CHUNK)],
                                  gsem[0]).wait()

        @pl.when(s < 2)
        def _():
            pltpu.make_async_copy(rows[0].at[pl.ds(0, ZROWS_BIG - nfull)],
                                  acc_sh.at[pl.ds(0, ZROWS_BIG - nfull)],
                                  gsem[0]).wait()

        @pl.when(s >= 2)
        def _():
            pltpu.make_async_copy(rows[0].at[pl.ds(0, ZROWS_SMALL - nfull)],
                                  acc_sh.at[pl.ds(0, ZROWS_SMALL - nfull)],
                                  gsem[0]).wait()

    plsc.subcore_barrier()

    ebase = wid * EPW

    def _idx(t, b):
        pltpu.async_copy(src_hbm.at[pl.ds(ebase + t * CHUNK, CHUNK)],
                         srcb[b], isem[b])
        pltpu.async_copy(dst_hbm.at[pl.ds(ebase + t * CHUNK, CHUNK)],
                         dstb[b], isem[b])

    def _iwait(b):
        pltpu.make_async_copy(src_hbm.at[pl.ds(0, CHUNK)], srcb[b],
                              isem[b]).wait()
        pltpu.make_async_copy(dst_hbm.at[pl.ds(0, CHUNK)], dstb[b],
                              isem[b]).wait()

    def _gather(b):
        pltpu.async_copy(x_hbm.at[srcb[b]], rows[b], gsem[b])

    def _gwait(b):
        pltpu.make_async_copy(x_hbm.at[pl.ds(0, CHUNK)], rows[b],
                              gsem[b]).wait()

    def _scat(b):
        pltpu.async_copy(rows[b], acc_sh.at[dstb[b]], ssem[b], add=True)

    def _swait(b):
        pltpu.make_async_copy(rows[b], acc_sh.at[pl.ds(0, CHUNK)],
                              ssem[b]).wait()

    for t in range(3):
        _idx(t, t)
    for t in range(2):
        _iwait(t)
        _gather(t)
    _gwait(0)
    _scat(0)
    _idx(3, 3)
    _iwait(2)
    _gather(2)

    def _group(g, carry):
        for j in range(NBUF):
            b = (1 + j) % NBUF
            _gwait(b)
            _swait((b + 3) % NBUF)
            _scat(b)
            _idx_dyn = g * NBUF + j
            pltpu.async_copy(
                src_hbm.at[pl.ds(ebase + (_idx_dyn + 4) * CHUNK, CHUNK)],
                srcb[(b + 3) % NBUF], isem[(b + 3) % NBUF])
            pltpu.async_copy(
                dst_hbm.at[pl.ds(ebase + (_idx_dyn + 4) * CHUNK, CHUNK)],
                dstb[(b + 3) % NBUF], isem[(b + 3) % NBUF])
            _iwait((b + 2) % NBUF)
            _gather((b + 2) % NBUF)
        return carry

    lax.fori_loop(0, (NCHUNK - 5) // NBUF, _group, 0)

    for t in range(NCHUNK - 4, NCHUNK):
        b = t % NBUF
        _gwait(b)
        _swait((b + 3) % NBUF)
        _scat(b)
        if t + 3 < NCHUNK:
            _idx(t + 3, (b + 3) % NBUF)
        if t + 2 < NCHUNK:
            _iwait((b + 2) % NBUF)
            _gather((b + 2) % NBUF)
    _swait((NCHUNK - 1) % NBUF)
    plsc.subcore_barrier()

    @pl.when(s < 2)
    def _():
        d0 = s * ZROWS_BIG
        pltpu.sync_copy(acc_sh.at[pl.ds(d0, ZROWS_BIG)],
                        out_hbm.at[c, pl.ds(d0, ZROWS_BIG)])

    @pl.when(s >= 2)
    def _():
        d0 = 2 * ZROWS_BIG + (s - 2) * ZROWS_SMALL
        pltpu.sync_copy(acc_sh.at[pl.ds(d0, ZROWS_SMALL)],
                        out_hbm.at[c, pl.ds(d0, ZROWS_SMALL)])


BLK = 2000
NB = N // BLK


def _emb_body(x_ref, w_ref, b_ref, o_ref):
    o_ref[...] = jnp.maximum(
        jnp.dot(x_ref[...], w_ref[...], preferred_element_type=jnp.float32)
        + b_ref[...], 0.0)


def _layer_body(p_ref, w1_ref, b1_ref, g_ref, beta_ref,
                w2_ref, b2_ref, batch_ref, xo_ref, pool_ref, y_scr, st_scr):
    ph = pl.program_id(0)
    i = pl.program_id(1)
    _layer_phase(ph, i, p_ref, w1_ref, b1_ref, g_ref, beta_ref,
                 w2_ref, b2_ref, batch_ref, xo_ref, pool_ref, y_scr, st_scr)


def _layer_head_body(p_ref, w1_ref, b1_ref, g_ref, beta_ref,
                     w2_ref, b2_ref, batch_ref, p0_ref, p1_ref,
                     fc1w_ref, fc1b_ref, fc2w_ref, fc2b_ref,
                     fcw_ref, fcb_ref, xo_ref, pool_ref, c_ref, h2_ref,
                     y_scr, st_scr):
    ph = pl.program_id(0)
    i = pl.program_id(1)
    _layer_phase(ph, i, p_ref, w1_ref, b1_ref, g_ref, beta_ref,
                 w2_ref, b2_ref, batch_ref, xo_ref, pool_ref, y_scr, st_scr)

    @pl.when(jnp.logical_and(ph == 1, i == NB - 1))
    def _():
        h1 = jnp.maximum(
            jnp.dot(p0_ref[...], fc1w_ref[0:H, :],
                    preferred_element_type=jnp.float32)
            + jnp.dot(p1_ref[...], fc1w_ref[H:2 * H, :],
                      preferred_element_type=jnp.float32)
            + jnp.dot(pool_ref[...], fc1w_ref[2 * H:3 * H, :],
                      preferred_element_type=jnp.float32)
            + fc1b_ref[...], 0.0)
        h2 = jnp.maximum(
            jnp.dot(h1, fc2w_ref[...], preferred_element_type=jnp.float32)
            + fc2b_ref[...], 0.0)
        h2_ref[...] = h2
        c_ref[...] = (jnp.dot(h2, fcw_ref[...],
                              preferred_element_type=jnp.float32)
                      + fcb_ref[...])


def _layer_phase(ph, i, p_ref, w1_ref, b1_ref, g_ref, beta_ref,
                 w2_ref, b2_ref, batch_ref, xo_ref, pool_ref, y_scr, st_scr):

    @pl.when(ph == 0)
    def _():
        z = p_ref[0] + p_ref[1]
        y = (jnp.dot(z, w1_ref[...], preferred_element_type=jnp.float32)
             + b1_ref[...])
        y_scr[pl.ds(i * BLK, BLK), :] = y
        ssum = jnp.sum(y, axis=0, keepdims=True)
        ssq = jnp.sum(y * y, axis=0, keepdims=True)

        @pl.when(i == 0)
        def _():
            st_scr[0:1, :] = ssum
            st_scr[1:2, :] = ssq

        @pl.when(i > 0)
        def _():
            st_scr[0:1, :] += ssum
            st_scr[1:2, :] += ssq

    @pl.when(ph == 1)
    def _():
        m = st_scr[0:1, :] * (1.0 / N)
        v = st_scr[1:2, :] * (1.0 / N) - m * m
        y = y_scr[pl.ds(i * BLK, BLK), :]
        yn = (y - m) * lax.rsqrt(v + 1e-5) * g_ref[...] + beta_ref[...]
        zr = jnp.maximum(yn, 0.0)
        xo = jnp.maximum(
            jnp.dot(zr, w2_ref[...], preferred_element_type=jnp.float32)
            + b2_ref[...], 0.0)
        xo_ref[...] = xo
        gid = lax.broadcasted_iota(jnp.int32, (G, BLK), 0)
        sel = (batch_ref[0] == gid).astype(jnp.float32)
        contrib = jnp.dot(sel, xo, preferred_element_type=jnp.float32)

        @pl.when(i == 0)
        def _():
            pool_ref[...] = contrib

        @pl.when(i > 0)
        def _():
            pool_ref[...] += contrib


def _head_body(h_ref, fc1w_ref, fc1b_ref, fc2w_ref, fc2b_ref,
               fcw_ref, fcb_ref, c_ref, h2_ref):
    h = jnp.maximum(
        jnp.dot(h_ref[...], fc1w_ref[...], preferred_element_type=jnp.float32)
        + fc1b_ref[...], 0.0)
    h2 = jnp.maximum(
        jnp.dot(h, fc2w_ref[...], preferred_element_type=jnp.float32)
        + fc2b_ref[...], 0.0)
    h2_ref[...] = h2
    c_ref[...] = (jnp.dot(h2, fcw_ref[...], preferred_element_type=jnp.float32)
                  + fcb_ref[...])


_emb_call = pl.pallas_call(
    _emb_body,
    grid=(NB,),
    in_specs=[
        pl.BlockSpec((BLK, D), lambda i: (i, 0)),
        pl.BlockSpec((D, H), lambda i: (0, 0)),
        pl.BlockSpec((1, H), lambda i: (0, 0)),
    ],
    out_specs=pl.BlockSpec((BLK, H), lambda i: (i, 0)),
    out_shape=jax.ShapeDtypeStruct((N, H), jnp.float32))

_layer_call = pl.pallas_call(
    _layer_body,
    grid=(2, NB),
    in_specs=[
        pl.BlockSpec((2, BLK, H),
                     lambda p, i: (0, jnp.where(p == 0, i, NB - 1), 0)),
        pl.BlockSpec((H, H), lambda p, i: (0, 0)),
        pl.BlockSpec((1, H), lambda p, i: (0, 0)),
        pl.BlockSpec((1, H), lambda p, i: (0, 0)),
        pl.BlockSpec((1, H), lambda p, i: (0, 0)),
        pl.BlockSpec((H, H), lambda p, i: (0, 0)),
        pl.BlockSpec((1, H), lambda p, i: (0, 0)),
        pl.BlockSpec((1, 1, BLK), lambda p, i: (jnp.where(p == 1, i, 0), 0, 0)),
    ],
    out_specs=(
        pl.BlockSpec((BLK, H), lambda p, i: (jnp.where(p == 1, i, 0), 0)),
        pl.BlockSpec((G, H), lambda p, i: (0, 0)),
    ),
    out_shape=(jax.ShapeDtypeStruct((N, H), jnp.float32),
               jax.ShapeDtypeStruct((G, H), jnp.float32)),
    scratch_shapes=[
        pltpu.VMEM((N, H), jnp.float32),
        pltpu.VMEM((8, H), jnp.float32),
    ])

_whole = lambda shape: pl.BlockSpec(shape, lambda p, i: tuple(0 for _ in shape))

_layer_head_call = pl.pallas_call(
    _layer_head_body,
    grid=(2, NB),
    in_specs=[
        pl.BlockSpec((2, BLK, H),
                     lambda p, i: (0, jnp.where(p == 0, i, NB - 1), 0)),
        _whole((H, H)),
        _whole((1, H)),
        _whole((1, H)),
        _whole((1, H)),
        _whole((H, H)),
        _whole((1, H)),
        pl.BlockSpec((1, 1, BLK), lambda p, i: (jnp.where(p == 1, i, 0), 0, 0)),
        _whole((G, H)),
        _whole((G, H)),
        _whole((H * L, H)),
        _whole((1, H)),
        _whole((H, H // 2)),
        _whole((1, H // 2)),
        _whole((H // 2, OUT)),
        _whole((1, OUT)),
    ],
    out_specs=(
        pl.BlockSpec((BLK, H), lambda p, i: (jnp.where(p == 1, i, 0), 0)),
        _whole((G, H)),
        _whole((G, OUT)),
        _whole((G, H // 2)),
    ),
    out_shape=(jax.ShapeDtypeStruct((N, H), jnp.float32),
               jax.ShapeDtypeStruct((G, H), jnp.float32),
               jax.ShapeDtypeStruct((G, OUT), jnp.float32),
               jax.ShapeDtypeStruct((G, H // 2), jnp.float32)),
    scratch_shapes=[
        pltpu.VMEM((N, H), jnp.float32),
        pltpu.VMEM((8, H), jnp.float32),
    ])


def kernel(x, edge_index, batch, emb_W, emb_b, l_W1, l_b1, l_g, l_beta,
           l_W2, l_b2, fc1_W, fc1_b, fc2_W, fc2_b, fc_W, fc_b):
    src = edge_index[0]
    dst = edge_index[1]
    batch2 = batch.reshape(NB, 1, BLK).astype(jnp.int32)

    xh = _emb_call(x, emb_W, emb_b.reshape(1, H))
    pools = []
    sc_aggregate = _make_sc_aggregate()
    for i in range(L - 1):
        part = sc_aggregate(xh, src, dst)
        xh, pool = _layer_call(
            part, l_W1[i], l_b1[i].reshape(1, H), l_g[i].reshape(1, H),
            l_beta[i].reshape(1, H), l_W2[i], l_b2[i].reshape(1, H), batch2)
        pools.append(pool)
    i = L - 1
    part = sc_aggregate(xh, src, dst)
    xh, _, c, h = _layer_head_call(
        part, l_W1[i], l_b1[i].reshape(1, H), l_g[i].reshape(1, H),
        l_beta[i].reshape(1, H), l_W2[i], l_b2[i].reshape(1, H), batch2,
        pools[0], pools[1], fc1_W, fc1_b.reshape(1, H), fc2_W,
        fc2_b.reshape(1, H // 2), fc_W, fc_b.reshape(1, OUT))
    return (c, h, xh)


# speedup vs baseline: 1.0198x; 1.0198x over previous
import functools

import jax
import jax.numpy as jnp
from jax import lax
from jax.experimental import pallas as pl
from jax.experimental.pallas import tpu as pltpu
from jax.experimental.pallas import tpu_sc as plsc

N = 10000
E = 320000
D = 128
H = 128
L = 3
G = 64
OUT = 10

NC = 2
NS = 16
NW = NC * NS
EPW = E // NW
CHUNK = 80
NCHUNK = EPW // CHUNK
NBUF = 4
ZROWS_BIG = 632
ZROWS_SMALL = 624

@functools.cache
def _make_sc_aggregate():
    mesh = plsc.VectorSubcoreMesh(core_axis_name="c", subcore_axis_name="s")
    return functools.partial(
        pl.kernel,
        mesh=mesh,
        out_type=jax.ShapeDtypeStruct((NC, N, H), jnp.float32),
        scratch_types=(
            [pltpu.VMEM_SHARED((N, H), jnp.float32)]
            + [pltpu.VMEM((CHUNK, H), jnp.float32)] * NBUF
            + [pltpu.VMEM((CHUNK,), jnp.int32)] * NBUF
            + [pltpu.VMEM((CHUNK,), jnp.int32)] * NBUF
            + [pltpu.SemaphoreType.DMA] * (3 * NBUF)
        ),
    )(_sc_aggregate_body)


def _sc_aggregate_body(x_hbm, src_hbm, dst_hbm, out_hbm,
                       acc_sh, *bufs_and_sems):
    rows = bufs_and_sems[:NBUF]
    srcb = bufs_and_sems[NBUF:2 * NBUF]
    dstb = bufs_and_sems[2 * NBUF:3 * NBUF]
    gsem = bufs_and_sems[3 * NBUF:4 * NBUF]
    ssem = bufs_and_sems[4 * NBUF:5 * NBUF]
    isem = bufs_and_sems[5 * NBUF:]
    c = lax.axis_index("c")
    s = lax.axis_index("s")
    wid = c * NS + s

    z16 = jnp.zeros((16,), jnp.float32)

    def _zero_body(i, carry):
        r = i // (H // 16)
        col = (i % (H // 16)) * 16
        rows[0][r, pl.ds(col, 16)] = z16
        return carry

    lax.fori_loop(0, CHUNK * (H // 16), _zero_body, 0)

    start_big = s * ZROWS_BIG
    start_small = 2 * ZROWS_BIG + (s - 2) * ZROWS_SMALL
    start = jnp.where(s < 2, start_big, start_small)
    nzfull = ZROWS_SMALL // CHUNK
    for k in range(nzfull):
        pltpu.async_copy(rows[0], acc_sh.at[pl.ds(start + k * CHUNK, CHUNK)],
                         gsem[0])
    nfull = nzfull * CHUNK

    @pl.when(s < 2)
    def _():
        pltpu.async_copy(rows[0].at[pl.ds(0, ZROWS_BIG - nfull)],
                        acc_sh.at[pl.ds(start + nfull, ZROWS_BIG - nfull)],
                        gsem[0])

    @pl.when(s >= 2)
    def _():
        pltpu.async_copy(rows[0].at[pl.ds(0, ZROWS_SMALL - nfull)],
                        acc_sh.at[pl.ds(start + nfull, ZROWS_SMALL - nfull)],
                        gsem[0])

    for k in range(nzfull):
        pltpu.make_async_copy(rows[0], acc_sh.at[pl.ds(0, CHUNK)],
                              gsem[0]).wait()

    @pl.when(s < 2)
    def _():
        pltpu.make_async_copy(rows[0].at[pl.ds(0, ZROWS_BIG - nfull)],
                              acc_sh.at[pl.ds(0, ZROWS_BIG - nfull)],
                              gsem[0]).wait()

    @pl.when(s >= 2)
    def _():
        pltpu.make_async_copy(rows[0].at[pl.ds(0, ZROWS_SMALL - nfull)],
                              acc_sh.at[pl.ds(0, ZROWS_SMALL - nfull)],
                              gsem[0]).wait()

    plsc.subcore_barrier()

    ebase = wid * EPW

    def _idx(t, b):
        pltpu.async_copy(src_hbm.at[pl.ds(ebase + t * CHUNK, CHUNK)],
                         srcb[b], isem[b])
        pltpu.async_copy(dst_hbm.at[pl.ds(ebase + t * CHUNK, CHUNK)],
                         dstb[b], isem[b])

    def _iwait(b):
        pltpu.make_async_copy(src_hbm.at[pl.ds(0, CHUNK)], srcb[b],
                              isem[b]).wait()
        pltpu.make_async_copy(dst_hbm.at[pl.ds(0, CHUNK)], dstb[b],
                              isem[b]).wait()

    def _gather(b):
        pltpu.async_copy(x_hbm.at[srcb[b]], rows[b], gsem[b])

    def _gwait(b):
        pltpu.make_async_copy(x_hbm.at[pl.ds(0, CHUNK)], rows[b],
                              gsem[b]).wait()

    def _scat(b):
        pltpu.async_copy(rows[b], acc_sh.at[dstb[b]], ssem[b], add=True)

    def _swait(b):
        pltpu.make_async_copy(rows[b], acc_sh.at[pl.ds(0, CHUNK)],
                              ssem[b]).wait()

    for t in range(3):
        _idx(t, t)
    for t in range(2):
        _iwait(t)
        _gather(t)
    _gwait(0)
    _scat(0)
    _idx(3, 3)
    _iwait(2)
    _gather(2)

    def _group(g, carry):
        for j in range(NBUF):
            b = (1 + j) % NBUF
            _gwait(b)
            _swait((b + 3) % NBUF)
            _scat(b)
            _idx_dyn = g * NBUF + j
            pltpu.async_copy(
                src_hbm.at[pl.ds(ebase + (_idx_dyn + 4) * CHUNK, CHUNK)],
                srcb[(b + 3) % NBUF], isem[(b + 3) % NBUF])
            pltpu.async_copy(
                dst_hbm.at[pl.ds(ebase + (_idx_dyn + 4) * CHUNK, CHUNK)],
                dstb[(b + 3) % NBUF], isem[(b + 3) % NBUF])
            _iwait((b + 2) % NBUF)
            _gather((b + 2) % NBUF)
        return carry

    lax.fori_loop(0, (NCHUNK - 5) // NBUF, _group, 0)

    for t in range(NCHUNK - 4, NCHUNK):
        b = t % NBUF
        _gwait(b)
        _swait((b + 3) % NBUF)
        _scat(b)
        if t + 3 < NCHUNK:
            _idx(t + 3, (b + 3) % NBUF)
        if t + 2 < NCHUNK:
            _iwait((b + 2) % NBUF)
            _gather((b + 2) % NBUF)
    _swait((NCHUNK - 1) % NBUF)
    plsc.subcore_barrier()

    @pl.when(s < 2)
    def _():
        d0 = s * ZROWS_BIG
        pltpu.sync_copy(acc_sh.at[pl.ds(d0, ZROWS_BIG)],
                        out_hbm.at[c, pl.ds(d0, ZROWS_BIG)])

    @pl.when(s >= 2)
    def _():
        d0 = 2 * ZROWS_BIG + (s - 2) * ZROWS_SMALL
        pltpu.sync_copy(acc_sh.at[pl.ds(d0, ZROWS_SMALL)],
                        out_hbm.at[c, pl.ds(d0, ZROWS_SMALL)])


BLK = 2000
NB = N // BLK


def _emb_body(x_ref, w_ref, b_ref, o_ref):
    o_ref[...] = jnp.maximum(
        jnp.dot(x_ref[...], w_ref[...], preferred_element_type=jnp.float32)
        + b_ref[...], 0.0)


def _layer_body(p_ref, x_ref, w1_ref, b1_ref, g_ref, beta_ref,
                w2_ref, b2_ref, batch_ref, xo_ref, pool_ref, y_scr, st_scr):
    ph = pl.program_id(0)
    i = pl.program_id(1)
    _layer_phase(ph, i, p_ref, x_ref, w1_ref, b1_ref, g_ref, beta_ref,
                 w2_ref, b2_ref, batch_ref, xo_ref, pool_ref, y_scr, st_scr)


def _layer_head_body(p_ref, x_ref, w1_ref, b1_ref, g_ref, beta_ref,
                     w2_ref, b2_ref, batch_ref, p0_ref, p1_ref,
                     fc1w_ref, fc1b_ref, fc2w_ref, fc2b_ref,
                     fcw_ref, fcb_ref, xo_ref, pool_ref, c_ref, h2_ref,
                     y_scr, st_scr):
    ph = pl.program_id(0)
    i = pl.program_id(1)
    _layer_phase(ph, i, p_ref, x_ref, w1_ref, b1_ref, g_ref, beta_ref,
                 w2_ref, b2_ref, batch_ref, xo_ref, pool_ref, y_scr, st_scr)

    @pl.when(jnp.logical_and(ph == 1, i == NB - 1))
    def _():
        h1 = jnp.maximum(
            jnp.dot(p0_ref[...], fc1w_ref[0:H, :],
                    preferred_element_type=jnp.float32)
            + jnp.dot(p1_ref[...], fc1w_ref[H:2 * H, :],
                      preferred_element_type=jnp.float32)
            + jnp.dot(pool_ref[...], fc1w_ref[2 * H:3 * H, :],
                      preferred_element_type=jnp.float32)
            + fc1b_ref[...], 0.0)
        h2 = jnp.maximum(
            jnp.dot(h1, fc2w_ref[...], preferred_element_type=jnp.float32)
            + fc2b_ref[...], 0.0)
        h2_ref[...] = h2
        c_ref[...] = (jnp.dot(h2, fcw_ref[...],
                              preferred_element_type=jnp.float32)
                      + fcb_ref[...])


def _layer_phase(ph, i, p_ref, x_ref, w1_ref, b1_ref, g_ref, beta_ref,
                 w2_ref, b2_ref, batch_ref, xo_ref, pool_ref, y_scr, st_scr):

    @pl.when(ph == 0)
    def _():
        z = p_ref[0] + p_ref[1] + x_ref[...]
        y = (jnp.dot(z, w1_ref[...], preferred_element_type=jnp.float32)
             + b1_ref[...])
        y_scr[pl.ds(i * BLK, BLK), :] = y
        ssum = jnp.sum(y, axis=0, keepdims=True)
        ssq = jnp.sum(y * y, axis=0, keepdims=True)

        @pl.when(i == 0)
        def _():
            st_scr[0:1, :] = ssum
            st_scr[1:2, :] = ssq

        @pl.when(i > 0)
        def _():
            st_scr[0:1, :] += ssum
            st_scr[1:2, :] += ssq

    @pl.when(ph == 1)
    def _():
        m = st_scr[0:1, :] * (1.0 / N)
        v = st_scr[1:2, :] * (1.0 / N) - m * m
        y = y_scr[pl.ds(i * BLK, BLK), :]
        yn = (y - m) * lax.rsqrt(v + 1e-5) * g_ref[...] + beta_ref[...]
        zr = jnp.maximum(yn, 0.0)
        xo = jnp.maximum(
            jnp.dot(zr, w2_ref[...], preferred_element_type=jnp.float32)
            + b2_ref[...], 0.0)
        xo_ref[...] = xo
        gid = lax.broadcasted_iota(jnp.int32, (G, BLK), 0)
        sel = (batch_ref[0] == gid).astype(jnp.float32)
        contrib = jnp.dot(sel, xo, preferred_element_type=jnp.float32)

        @pl.when(i == 0)
        def _():
            pool_ref[...] = contrib

        @pl.when(i > 0)
        def _():
            pool_ref[...] += contrib


def _head_body(h_ref, fc1w_ref, fc1b_ref, fc2w_ref, fc2b_ref,
               fcw_ref, fcb_ref, c_ref, h2_ref):
    h = jnp.maximum(
        jnp.dot(h_ref[...], fc1w_ref[...], preferred_element_type=jnp.float32)
        + fc1b_ref[...], 0.0)
    h2 = jnp.maximum(
        jnp.dot(h, fc2w_ref[...], preferred_element_type=jnp.float32)
        + fc2b_ref[...], 0.0)
    h2_ref[...] = h2
    c_ref[...] = (jnp.dot(h2, fcw_ref[...], preferred_element_type=jnp.float32)
                  + fcb_ref[...])


_emb_call = pl.pallas_call(
    _emb_body,
    grid=(NB,),
    in_specs=[
        pl.BlockSpec((BLK, D), lambda i: (i, 0)),
        pl.BlockSpec((D, H), lambda i: (0, 0)),
        pl.BlockSpec((1, H), lambda i: (0, 0)),
    ],
    out_specs=pl.BlockSpec((BLK, H), lambda i: (i, 0)),
    out_shape=jax.ShapeDtypeStruct((N, H), jnp.float32))

_layer_call = pl.pallas_call(
    _layer_body,
    grid=(2, NB),
    in_specs=[
        pl.BlockSpec((2, BLK, H),
                     lambda p, i: (0, jnp.where(p == 0, i, NB - 1), 0)),
        pl.BlockSpec((BLK, H),
                     lambda p, i: (jnp.where(p == 0, i, NB - 1), 0)),
        pl.BlockSpec((H, H), lambda p, i: (0, 0)),
        pl.BlockSpec((1, H), lambda p, i: (0, 0)),
        pl.BlockSpec((1, H), lambda p, i: (0, 0)),
        pl.BlockSpec((1, H), lambda p, i: (0, 0)),
        pl.BlockSpec((H, H), lambda p, i: (0, 0)),
        pl.BlockSpec((1, H), lambda p, i: (0, 0)),
        pl.BlockSpec((1, 1, BLK), lambda p, i: (jnp.where(p == 1, i, 0), 0, 0)),
    ],
    out_specs=(
        pl.BlockSpec((BLK, H), lambda p, i: (jnp.where(p == 1, i, 0), 0)),
        pl.BlockSpec((G, H), lambda p, i: (0, 0)),
    ),
    out_shape=(jax.ShapeDtypeStruct((N, H), jnp.float32),
               jax.ShapeDtypeStruct((G, H), jnp.float32)),
    scratch_shapes=[
        pltpu.VMEM((N, H), jnp.float32),
        pltpu.VMEM((8, H), jnp.float32),
    ])

_whole = lambda shape: pl.BlockSpec(shape, lambda p, i: tuple(0 for _ in shape))

_layer_head_call = pl.pallas_call(
    _layer_head_body,
    grid=(2, NB),
    in_specs=[
        pl.BlockSpec((2, BLK, H),
                     lambda p, i: (0, jnp.where(p == 0, i, NB - 1), 0)),
        pl.BlockSpec((BLK, H),
                     lambda p, i: (jnp.where(p == 0, i, NB - 1), 0)),
        _whole((H, H)),
        _whole((1, H)),
        _whole((1, H)),
        _whole((1, H)),
        _whole((H, H)),
        _whole((1, H)),
        pl.BlockSpec((1, 1, BLK), lambda p, i: (jnp.where(p == 1, i, 0), 0, 0)),
        _whole((G, H)),
        _whole((G, H)),
        _whole((H * L, H)),
        _whole((1, H)),
        _whole((H, H // 2)),
        _whole((1, H // 2)),
        _whole((H // 2, OUT)),
        _whole((1, OUT)),
    ],
    out_specs=(
        pl.BlockSpec((BLK, H), lambda p, i: (jnp.where(p == 1, i, 0), 0)),
        _whole((G, H)),
        _whole((G, OUT)),
        _whole((G, H // 2)),
    ),
    out_shape=(jax.ShapeDtypeStruct((N, H), jnp.float32),
               jax.ShapeDtypeStruct((G, H), jnp.float32),
               jax.ShapeDtypeStruct((G, OUT), jnp.float32),
               jax.ShapeDtypeStruct((G, H // 2), jnp.float32)),
    scratch_shapes=[
        pltpu.VMEM((N, H), jnp.float32),
        pltpu.VMEM((8, H), jnp.float32),
    ])


def kernel(x, edge_index, batch, emb_W, emb_b, l_W1, l_b1, l_g, l_beta,
           l_W2, l_b2, fc1_W, fc1_b, fc2_W, fc2_b, fc_W, fc_b):
    src = edge_index[0]
    dst = edge_index[1]
    batch2 = batch.reshape(NB, 1, BLK).astype(jnp.int32)

    xh = _emb_call(x, emb_W, emb_b.reshape(1, H))
    pools = []
    sc_aggregate = _make_sc_aggregate()
    for i in range(L - 1):
        part = sc_aggregate(xh, src, dst)
        xh, pool = _layer_call(
            part, xh, l_W1[i], l_b1[i].reshape(1, H), l_g[i].reshape(1, H),
            l_beta[i].reshape(1, H), l_W2[i], l_b2[i].reshape(1, H), batch2)
        pools.append(pool)
    i = L - 1
    part = sc_aggregate(xh, src, dst)
    xh, _, c, h = _layer_head_call(
        part, xh, l_W1[i], l_b1[i].reshape(1, H), l_g[i].reshape(1, H),
        l_beta[i].reshape(1, H), l_W2[i], l_b2[i].reshape(1, H), batch2,
        pools[0], pools[1], fc1_W, fc1_b.reshape(1, H), fc2_W,
        fc2_b.reshape(1, H // 2), fc_W, fc_b.reshape(1, OUT))
    return (c, h, xh)
